# baseline (speedup 1.0000x reference)
import jax
import jax.numpy as jnp
from jax.experimental import pallas as pl


def _split_body(z1_ref, z2_ref, a_ref, b_ref):
    a_ref[...] = z1_ref[...]
    b_ref[...] = z2_ref[...]


def kernel(z):
    n, c, h, w = z.shape
    ch = c // 2
    cols = ch * h * w
    rows = cols // 128
    z3 = z.reshape(n, 2 * rows, 128)

    out1, out2 = pl.pallas_call(
        _split_body,
        grid=(n,),
        in_specs=[
            pl.BlockSpec((1, rows, 128), lambda i: (i, 0, 0)),
            pl.BlockSpec((1, rows, 128), lambda i: (i, 1, 0)),
        ],
        out_specs=[
            pl.BlockSpec((1, rows, 128), lambda i: (i, 0, 0)),
            pl.BlockSpec((1, rows, 128), lambda i: (i, 0, 0)),
        ],
        out_shape=[
            jax.ShapeDtypeStruct((n, rows, 128), z.dtype),
            jax.ShapeDtypeStruct((n, rows, 128), z.dtype),
        ],
    )(z3, z3)

    z1 = out1.reshape(n, ch, h, w)
    z2 = out2.reshape(n, ch, h, w)
    log_det = jnp.zeros((), z.dtype)
    return (z1, z2, log_det)
